# Initial kernel scaffold; baseline (speedup 1.0000x reference)
import jax
import jax.numpy as jnp
from jax.experimental import pallas as pl


def kernel(input_ids, decoder_input_ids, table):
    raise NotImplementedError("write your pallas kernel here")


# speedup vs baseline: 1.5334x; 1.5334x over previous
import functools

import jax
import jax.numpy as jnp
from jax import lax
from jax.experimental import pallas as pl
from jax.experimental.pallas import tpu as pltpu
from jax.experimental.pallas import tpu_sc as plsc

_INFO = plsc.get_sparse_core_info()
_NC = _INFO.num_cores
_NS = _INFO.num_subcores
_NW = _NC * _NS

_CHUNK = 128


@functools.partial(jax.jit, static_argnums=(3, 4))
def _sc_gather(enc_idx, dec_idx, table, n_enc, n_dec):
    V, D = table.shape
    k_enc = enc_idx.shape[1]
    k_dec = dec_idx.shape[1]
    enc_per_w = k_enc * _CHUNK
    dec_per_w = k_dec * _CHUNK
    rows_per_w = enc_per_w + dec_per_w

    mesh = plsc.VectorSubcoreMesh(core_axis_name="c", subcore_axis_name="s")

    @functools.partial(
        pl.kernel,
        mesh=mesh,
        out_type=(
            jax.ShapeDtypeStruct((n_enc, D), jnp.float32),
            jax.ShapeDtypeStruct((n_dec, D), jnp.float32),
        ),
        scratch_types=[
            pltpu.VMEM((k_enc + k_dec, _CHUNK), jnp.int32),
            pltpu.VMEM((rows_per_w, D), jnp.float32),
            pltpu.SemaphoreType.DMA,
        ],
    )
    def k(enc_hbm, dec_hbm, table_hbm, out_enc, out_dec, idx_v, rows_v, sem):
        wid = lax.axis_index("s") * _NC + lax.axis_index("c")
        pltpu.sync_copy(enc_hbm.at[wid], idx_v.at[pl.ds(0, k_enc)])
        pltpu.sync_copy(dec_hbm.at[wid], idx_v.at[pl.ds(k_enc, k_dec)])
        copies = []
        for j in range(k_enc + k_dec):
            copies.append(
                pltpu.async_copy(
                    table_hbm.at[idx_v.at[j]],
                    rows_v.at[pl.ds(j * _CHUNK, _CHUNK)],
                    sem,
                )
            )
        for c in copies:
            c.wait()
        pltpu.sync_copy(
            rows_v.at[pl.ds(0, enc_per_w)],
            out_enc.at[pl.ds(wid * enc_per_w, enc_per_w)],
        )
        pltpu.sync_copy(
            rows_v.at[pl.ds(enc_per_w, dec_per_w)],
            out_dec.at[pl.ds(wid * dec_per_w, dec_per_w)],
        )

    return k(enc_idx, dec_idx, table)


def kernel(input_ids, decoder_input_ids, table):
    B, S_enc = input_ids.shape
    _, S_dec = decoder_input_ids.shape
    D = table.shape[1]
    n_enc = B * S_enc
    n_dec = B * S_dec
    enc_idx = input_ids.astype(jnp.int32).reshape(_NW, n_enc // (_NW * _CHUNK), _CHUNK)
    dec_idx = decoder_input_ids.astype(jnp.int32).reshape(
        _NW, n_dec // (_NW * _CHUNK), _CHUNK
    )
    out_enc, out_dec = _sc_gather(enc_idx, dec_idx, table, n_enc, n_dec)
    return (
        out_enc.reshape(B, S_enc, D),
        out_dec.reshape(B, S_dec, D),
    )
